# baseline (speedup 1.0000x reference)
import jax
import jax.numpy as jnp
from jax import lax
from jax.experimental import pallas as pl
from jax.experimental.pallas import tpu as pltpu
from jax.experimental.pallas import tpu_sc as plsc

NC, NS, L = 2, 16, 16
NW = NC * NS
N = 100000
D = 256
CA = 3136
CT = N - (NW - 1) * CA
RCH = 800
NCH = N // RCH
KMAX = (NCH + NW - 1) // NW
RNAN = 160

_mesh = plsc.VectorSubcoreMesh(
    core_axis_name="c", subcore_axis_name="s", num_cores=NC, num_subcores=NS
)


def _counts_body(ent_hbm, gra_hbm, out_hbm, e_v, g_v, p_v):
    wid = lax.axis_index("s") * NC + lax.axis_index("c")
    base = wid * CA

    @pl.when(wid < NW - 1)
    def _():
        pltpu.sync_copy(ent_hbm.at[pl.ds(base, CA)], e_v)
        pltpu.sync_copy(gra_hbm.at[pl.ds(base, CA)], g_v.at[pl.ds(L, CA)])

    @pl.when(wid == NW - 1)
    def _():
        pltpu.sync_copy(ent_hbm.at[pl.ds(base, CT)], e_v.at[pl.ds(0, CT)])
        pltpu.sync_copy(
            gra_hbm.at[pl.ds(base, CT)], g_v.at[pl.ds(L, CT)]
        )

    @pl.when(wid > 0)
    def _():
        pltpu.sync_copy(gra_hbm.at[pl.ds(base - L, L)], g_v.at[pl.ds(0, L)])

    @pl.when(wid == 0)
    def _():
        g_v[pl.ds(0, L)] = jnp.full((L,), -1, jnp.int32)

    zero = jnp.zeros((L,), jnp.int32)
    one = jnp.ones((L,), jnp.int32)
    nv = jnp.where(wid < NW - 1, CA // L, CT // L)

    def step(i, carry):
        nt, nb = carry
        e = e_v[pl.ds(i * L, L)]
        cur = g_v[pl.ds(L + i * L, L)]
        prev = g_v[pl.ds(L - 1 + i * L, L)]
        nt = nt + jnp.where(e == 0, one, zero)
        nb = nb + jnp.where(cur != prev, one, zero)
        return nt, nb

    nt, nb = lax.fori_loop(0, nv, step, (zero, zero))
    p_v[0, :] = nt
    p_v[1, :] = nb
    pltpu.sync_copy(p_v, out_hbm.at[wid])


_counts = pl.kernel(
    _counts_body,
    out_type=jax.ShapeDtypeStruct((NW, 2, L), jnp.int32),
    mesh=_mesh,
    scratch_types=[
        pltpu.VMEM((CA,), jnp.int32),
        pltpu.VMEM((CA + L,), jnp.int32),
        pltpu.VMEM((2, L), jnp.int32),
    ],
)


def _select_body(x_hbm, parts_hbm, out_hbm, nanbuf, p_all, sem):
    wid = lax.axis_index("s") * NC + lax.axis_index("c")
    nk = jnp.where(wid < NCH % NW, KMAX, KMAX - 1)

    pltpu.sync_copy(parts_hbm, p_all)

    def red(i, d):
        return d + p_all[i, 0, :] - p_all[i, 1, :]

    diff = lax.fori_loop(0, NW, red, jnp.zeros((L,), jnp.int32))
    tot = jnp.int32(0)
    for q in range(L):
        tot = tot + diff[q]
    bad = tot != 0
    nanv = jnp.full((L,), jnp.nan, jnp.float32)

    row0 = [(wid + k * NW) * RCH for k in range(KMAX)]
    ok_cp = [
        pltpu.make_async_copy(
            x_hbm.at[pl.ds(row0[k], RCH)], out_hbm.at[pl.ds(row0[k], RCH)], sem
        )
        for k in range(KMAX)
    ]
    nan_cp = [
        [
            pltpu.make_async_copy(
                nanbuf, out_hbm.at[pl.ds(row0[k] + j * RNAN, RNAN)], sem
            )
            for j in range(RCH // RNAN)
        ]
        for k in range(KMAX)
    ]

    @pl.when(~bad)
    def _():
        for k in range(KMAX):

            @pl.when(k < nk)
            def _(k=k):
                ok_cp[k].start()

        for k in range(KMAX):

            @pl.when(k < nk)
            def _(k=k):
                ok_cp[k].wait()

    @pl.when(bad)
    def _():
        def nan_row(j, _):
            for q in range(D // L):
                nanbuf[j, pl.ds(q * L, L)] = nanv
            return 0

        lax.fori_loop(0, RNAN, nan_row, 0)
        for k in range(KMAX):

            @pl.when(k < nk)
            def _(k=k):
                for j in range(RCH // RNAN):
                    nan_cp[k][j].start()

        for k in range(KMAX):

            @pl.when(k < nk)
            def _(k=k):
                for j in range(RCH // RNAN):
                    nan_cp[k][j].wait()


_select = pl.kernel(
    _select_body,
    out_type=jax.ShapeDtypeStruct((N, D), jnp.float32),
    mesh=_mesh,
    scratch_types=[
        pltpu.VMEM((RNAN, D), jnp.float32),
        pltpu.VMEM((NW, 2, L), jnp.int32),
        pltpu.SemaphoreType.DMA,
    ],
)


def kernel(x_e, graph_ids, entity_ids):
    graph_ids = graph_ids.astype(jnp.int32)
    entity_ids = entity_ids.astype(jnp.int32)
    parts = _counts(entity_ids, graph_ids)
    return _select(x_e, parts)


# speedup vs baseline: 31.7680x; 31.7680x over previous
import jax
import jax.numpy as jnp
from jax import lax
from jax.experimental import pallas as pl
from jax.experimental.pallas import tpu as pltpu
from jax.experimental.pallas import tpu_sc as plsc

NC, NS, L = 2, 16, 16
NW = NC * NS
N = 100000
D = 256
CA = 3136
CT = N - (NW - 1) * CA
RCH = 160
NCH = N // RCH
KMAX = (NCH + NW - 1) // NW
NB = 3

_mesh = plsc.VectorSubcoreMesh(
    core_axis_name="c", subcore_axis_name="s", num_cores=NC, num_subcores=NS
)


def _counts_body(ent_hbm, gra_hbm, out_hbm, e_v, g_v, p_v):
    wid = lax.axis_index("s") * NC + lax.axis_index("c")
    base = wid * CA

    @pl.when(wid < NW - 1)
    def _():
        pltpu.sync_copy(ent_hbm.at[pl.ds(base, CA)], e_v)
        pltpu.sync_copy(gra_hbm.at[pl.ds(base, CA)], g_v.at[pl.ds(L, CA)])

    @pl.when(wid == NW - 1)
    def _():
        pltpu.sync_copy(ent_hbm.at[pl.ds(base, CT)], e_v.at[pl.ds(0, CT)])
        pltpu.sync_copy(gra_hbm.at[pl.ds(base, CT)], g_v.at[pl.ds(L, CT)])

    @pl.when(wid > 0)
    def _():
        pltpu.sync_copy(gra_hbm.at[pl.ds(base - L, L)], g_v.at[pl.ds(0, L)])

    @pl.when(wid == 0)
    def _():
        g_v[pl.ds(0, L)] = jnp.full((L,), -1, jnp.int32)

    zero = jnp.zeros((L,), jnp.int32)
    one = jnp.ones((L,), jnp.int32)
    nv = jnp.where(wid < NW - 1, CA // L, CT // L)

    def step(i, carry):
        nt, nb = carry
        e = e_v[pl.ds(i * L, L)]
        cur = g_v[pl.ds(L + i * L, L)]
        prev = g_v[pl.ds(L - 1 + i * L, L)]
        nt = nt + jnp.where(e == 0, one, zero)
        nb = nb + jnp.where(cur != prev, one, zero)
        return nt, nb

    nt, nb = lax.fori_loop(0, nv, step, (zero, zero))
    p_v[0, :] = nt
    p_v[1, :] = nb
    pltpu.sync_copy(p_v, out_hbm.at[wid])


_counts = pl.kernel(
    _counts_body,
    out_type=jax.ShapeDtypeStruct((NW, 2, L), jnp.int32),
    mesh=_mesh,
    scratch_types=[
        pltpu.VMEM((CA,), jnp.int32),
        pltpu.VMEM((CA + L,), jnp.int32),
        pltpu.VMEM((2, L), jnp.int32),
    ],
)


def _copy_body(x_hbm, out_hbm, bufs, *sems):
    insems, outsems = sems[:NB], sems[NB:]
    wid = lax.axis_index("s") * NC + lax.axis_index("c")
    nk = jnp.where(wid < NCH % NW, KMAX, KMAX - 1)

    row0 = [(wid + k * NW) * RCH for k in range(KMAX)]
    in_cp = [
        pltpu.make_async_copy(
            x_hbm.at[pl.ds(row0[k], RCH)], bufs.at[k % NB], insems[k % NB]
        )
        for k in range(KMAX)
    ]
    out_cp = [
        pltpu.make_async_copy(
            bufs.at[k % NB], out_hbm.at[pl.ds(row0[k], RCH)], outsems[k % NB]
        )
        for k in range(KMAX)
    ]

    in_cp[0].start()
    for k in range(KMAX):
        if k + 1 < KMAX:
            if k + 1 < NB:
                in_cp[k + 1].start()
            else:

                @pl.when(k + 1 < nk)
                def _(k=k):
                    out_cp[k + 1 - NB].wait()
                    in_cp[k + 1].start()

        @pl.when(k < nk)
        def _(k=k):
            in_cp[k].wait()
            out_cp[k].start()

    for k in range(max(0, KMAX - NB - 1), KMAX):

        @pl.when((k >= nk - NB) & (k < nk))
        def _(k=k):
            out_cp[k].wait()


_copy = pl.kernel(
    _copy_body,
    out_type=jax.ShapeDtypeStruct((N, D), jnp.float32),
    mesh=_mesh,
    scratch_types=[pltpu.VMEM((NB, RCH, D), jnp.float32)]
    + [pltpu.SemaphoreType.DMA] * (2 * NB),
)


def _nanfill_body(out_hbm, buf, sem):
    wid = lax.axis_index("s") * NC + lax.axis_index("c")
    nk = jnp.where(wid < NCH % NW, KMAX, KMAX - 1)
    nanv = jnp.full((L,), jnp.nan, jnp.float32)

    def nan_row(j, _):
        for q in range(D // L):
            buf[j, pl.ds(q * L, L)] = nanv
        return 0

    lax.fori_loop(0, RCH, nan_row, 0)
    cps = [
        pltpu.make_async_copy(
            buf, out_hbm.at[pl.ds((wid + k * NW) * RCH, RCH)], sem
        )
        for k in range(KMAX)
    ]
    for k in range(KMAX):

        @pl.when(k < nk)
        def _(k=k):
            cps[k].start()

    for k in range(KMAX):

        @pl.when(k < nk)
        def _(k=k):
            cps[k].wait()


_nanfill = pl.kernel(
    _nanfill_body,
    out_type=jax.ShapeDtypeStruct((N, D), jnp.float32),
    mesh=_mesh,
    scratch_types=[
        pltpu.VMEM((RCH, D), jnp.float32),
        pltpu.SemaphoreType.DMA,
    ],
)


def kernel(x_e, graph_ids, entity_ids):
    graph_ids = graph_ids.astype(jnp.int32)
    entity_ids = entity_ids.astype(jnp.int32)
    parts = _counts(entity_ids, graph_ids)
    rows = _copy(x_e)
    bad = jnp.sum(parts[:, 0, :]) != jnp.sum(parts[:, 1, :])
    return lax.cond(bad, lambda r: _nanfill(), lambda r: r, rows)


# speedup vs baseline: 33.1071x; 1.0422x over previous
import jax
import jax.numpy as jnp
from jax import lax
from jax.experimental import pallas as pl
from jax.experimental.pallas import tpu as pltpu
from jax.experimental.pallas import tpu_sc as plsc

NC, NS, L = 2, 16, 16
NW = NC * NS
N = 100000
D = 256
CA = 3136
CT = N - (NW - 1) * CA
RCH = 160
NCH = N // RCH
KMAX = (NCH + NW - 1) // NW
NB = 3

_mesh = plsc.VectorSubcoreMesh(
    core_axis_name="c", subcore_axis_name="s", num_cores=NC, num_subcores=NS
)


def _main_body(x_hbm, ent_hbm, gra_hbm, parts_hbm, out_hbm, bufs, e_v, g_v,
               p_v, *sems):
    insems, outsems = sems[:NB], sems[NB:]
    wid = lax.axis_index("s") * NC + lax.axis_index("c")
    nk = jnp.where(wid < NCH % NW, KMAX, KMAX - 1)

    row0 = [(wid + k * NW) * RCH for k in range(KMAX)]
    in_cp = [
        pltpu.make_async_copy(
            x_hbm.at[pl.ds(row0[k], RCH)], bufs.at[k % NB], insems[k % NB]
        )
        for k in range(KMAX)
    ]
    out_cp = [
        pltpu.make_async_copy(
            bufs.at[k % NB], out_hbm.at[pl.ds(row0[k], RCH)], outsems[k % NB]
        )
        for k in range(KMAX)
    ]

    in_cp[0].start()
    if NB >= 2:
        in_cp[1].start()

    base = wid * CA

    @pl.when(wid < NW - 1)
    def _():
        pltpu.sync_copy(ent_hbm.at[pl.ds(base, CA)], e_v)
        pltpu.sync_copy(gra_hbm.at[pl.ds(base, CA)], g_v.at[pl.ds(L, CA)])

    @pl.when(wid == NW - 1)
    def _():
        pltpu.sync_copy(ent_hbm.at[pl.ds(base, CT)], e_v.at[pl.ds(0, CT)])
        pltpu.sync_copy(gra_hbm.at[pl.ds(base, CT)], g_v.at[pl.ds(L, CT)])

    @pl.when(wid > 0)
    def _():
        pltpu.sync_copy(gra_hbm.at[pl.ds(base - L, L)], g_v.at[pl.ds(0, L)])

    @pl.when(wid == 0)
    def _():
        g_v[pl.ds(0, L)] = jnp.full((L,), -1, jnp.int32)

    zero = jnp.zeros((L,), jnp.int32)
    one = jnp.ones((L,), jnp.int32)
    nv = jnp.where(wid < NW - 1, CA // L, CT // L)

    def step(i, d):
        e = e_v[pl.ds(i * L, L)]
        cur = g_v[pl.ds(L + i * L, L)]
        prev = g_v[pl.ds(L - 1 + i * L, L)]
        d = d + jnp.where(e == 0, one, zero)
        return d - jnp.where(cur != prev, one, zero)

    p_v[...] = lax.fori_loop(0, nv, step, zero)
    pltpu.sync_copy(p_v, parts_hbm.at[wid])

    for k in range(KMAX):
        if k + 1 < KMAX and k + 1 >= 2:
            if k + 1 < NB:
                in_cp[k + 1].start()
            else:

                @pl.when(k + 1 < nk)
                def _(k=k):
                    out_cp[k + 1 - NB].wait()
                    in_cp[k + 1].start()

        @pl.when(k < nk)
        def _(k=k):
            in_cp[k].wait()
            out_cp[k].start()

    for k in range(max(0, KMAX - NB - 1), KMAX):

        @pl.when((k >= nk - NB) & (k < nk))
        def _(k=k):
            out_cp[k].wait()


_main = pl.kernel(
    _main_body,
    out_type=(
        jax.ShapeDtypeStruct((NW, L), jnp.int32),
        jax.ShapeDtypeStruct((N, D), jnp.float32),
    ),
    mesh=_mesh,
    scratch_types=[
        pltpu.VMEM((NB, RCH, D), jnp.float32),
        pltpu.VMEM((CA,), jnp.int32),
        pltpu.VMEM((CA + L,), jnp.int32),
        pltpu.VMEM((L,), jnp.int32),
    ]
    + [pltpu.SemaphoreType.DMA] * (2 * NB),
)


def _nanfill_body(out_hbm, buf, sem):
    wid = lax.axis_index("s") * NC + lax.axis_index("c")
    nk = jnp.where(wid < NCH % NW, KMAX, KMAX - 1)
    nanv = jnp.full((L,), jnp.nan, jnp.float32)

    def nan_row(j, _):
        for q in range(D // L):
            buf[j, pl.ds(q * L, L)] = nanv
        return 0

    lax.fori_loop(0, RCH, nan_row, 0)
    cps = [
        pltpu.make_async_copy(
            buf, out_hbm.at[pl.ds((wid + k * NW) * RCH, RCH)], sem
        )
        for k in range(KMAX)
    ]
    for k in range(KMAX):

        @pl.when(k < nk)
        def _(k=k):
            cps[k].start()

    for k in range(KMAX):

        @pl.when(k < nk)
        def _(k=k):
            cps[k].wait()


_nanfill = pl.kernel(
    _nanfill_body,
    out_type=jax.ShapeDtypeStruct((N, D), jnp.float32),
    mesh=_mesh,
    scratch_types=[
        pltpu.VMEM((RCH, D), jnp.float32),
        pltpu.SemaphoreType.DMA,
    ],
)


def kernel(x_e, graph_ids, entity_ids):
    graph_ids = graph_ids.astype(jnp.int32)
    entity_ids = entity_ids.astype(jnp.int32)
    parts, rows = _main(x_e, entity_ids, graph_ids)
    bad = jnp.sum(parts) != 0
    return lax.cond(bad, lambda r: _nanfill(), lambda r: r, rows)


# speedup vs baseline: 33.8241x; 1.0217x over previous
import jax
import jax.numpy as jnp
from jax import lax
from jax.experimental import pallas as pl
from jax.experimental.pallas import tpu as pltpu
from jax.experimental.pallas import tpu_sc as plsc

NC, NS, L = 2, 16, 16
NW = NC * NS
N = 100000
D = 256
CA = 3136
CT = N - (NW - 1) * CA
RCH = 200
NCH = N // RCH
KMAX = (NCH + NW - 1) // NW
NB = 2

_mesh = plsc.VectorSubcoreMesh(
    core_axis_name="c", subcore_axis_name="s", num_cores=NC, num_subcores=NS
)


def _main_body(x_hbm, ent_hbm, gra_hbm, parts_hbm, out_hbm, bufs, e_v, g_v,
               p_v, *sems):
    insems, outsems = sems[:NB], sems[NB:]
    wid = lax.axis_index("s") * NC + lax.axis_index("c")
    nk = jnp.where(wid < NCH % NW, KMAX, KMAX - 1)

    row0 = [(wid + k * NW) * RCH for k in range(KMAX)]
    in_cp = [
        pltpu.make_async_copy(
            x_hbm.at[pl.ds(row0[k], RCH)], bufs.at[k % NB], insems[k % NB]
        )
        for k in range(KMAX)
    ]
    out_cp = [
        pltpu.make_async_copy(
            bufs.at[k % NB], out_hbm.at[pl.ds(row0[k], RCH)], outsems[k % NB]
        )
        for k in range(KMAX)
    ]

    in_cp[0].start()
    if NB >= 2:
        in_cp[1].start()

    base = wid * CA

    @pl.when(wid < NW - 1)
    def _():
        pltpu.sync_copy(ent_hbm.at[pl.ds(base, CA)], e_v)
        pltpu.sync_copy(gra_hbm.at[pl.ds(base, CA)], g_v.at[pl.ds(L, CA)])

    @pl.when(wid == NW - 1)
    def _():
        pltpu.sync_copy(ent_hbm.at[pl.ds(base, CT)], e_v.at[pl.ds(0, CT)])
        pltpu.sync_copy(gra_hbm.at[pl.ds(base, CT)], g_v.at[pl.ds(L, CT)])

    @pl.when(wid > 0)
    def _():
        pltpu.sync_copy(gra_hbm.at[pl.ds(base - L, L)], g_v.at[pl.ds(0, L)])

    @pl.when(wid == 0)
    def _():
        g_v[pl.ds(0, L)] = jnp.full((L,), -1, jnp.int32)

    zero = jnp.zeros((L,), jnp.int32)
    one = jnp.ones((L,), jnp.int32)
    nv = jnp.where(wid < NW - 1, CA // L, CT // L)

    def step(i, d):
        e = e_v[pl.ds(i * L, L)]
        cur = g_v[pl.ds(L + i * L, L)]
        prev = g_v[pl.ds(L - 1 + i * L, L)]
        d = d + jnp.where(e == 0, one, zero)
        return d - jnp.where(cur != prev, one, zero)

    p_v[...] = lax.fori_loop(0, nv, step, zero)
    pltpu.sync_copy(p_v, parts_hbm.at[wid])

    for k in range(KMAX):
        if k + 1 < KMAX and k + 1 >= 2:
            if k + 1 < NB:
                in_cp[k + 1].start()
            else:

                @pl.when(k + 1 < nk)
                def _(k=k):
                    out_cp[k + 1 - NB].wait()
                    in_cp[k + 1].start()

        @pl.when(k < nk)
        def _(k=k):
            in_cp[k].wait()
            out_cp[k].start()

    for k in range(max(0, KMAX - NB - 1), KMAX):

        @pl.when((k >= nk - NB) & (k < nk))
        def _(k=k):
            out_cp[k].wait()


_main = pl.kernel(
    _main_body,
    out_type=(
        jax.ShapeDtypeStruct((NW, L), jnp.int32),
        jax.ShapeDtypeStruct((N, D), jnp.float32),
    ),
    mesh=_mesh,
    scratch_types=[
        pltpu.VMEM((NB, RCH, D), jnp.float32),
        pltpu.VMEM((CA,), jnp.int32),
        pltpu.VMEM((CA + L,), jnp.int32),
        pltpu.VMEM((L,), jnp.int32),
    ]
    + [pltpu.SemaphoreType.DMA] * (2 * NB),
)


def _nanfill_body(out_hbm, buf, sem):
    wid = lax.axis_index("s") * NC + lax.axis_index("c")
    nk = jnp.where(wid < NCH % NW, KMAX, KMAX - 1)
    nanv = jnp.full((L,), jnp.nan, jnp.float32)

    def nan_row(j, _):
        for q in range(D // L):
            buf[j, pl.ds(q * L, L)] = nanv
        return 0

    lax.fori_loop(0, RCH, nan_row, 0)
    cps = [
        pltpu.make_async_copy(
            buf, out_hbm.at[pl.ds((wid + k * NW) * RCH, RCH)], sem
        )
        for k in range(KMAX)
    ]
    for k in range(KMAX):

        @pl.when(k < nk)
        def _(k=k):
            cps[k].start()

    for k in range(KMAX):

        @pl.when(k < nk)
        def _(k=k):
            cps[k].wait()


_nanfill = pl.kernel(
    _nanfill_body,
    out_type=jax.ShapeDtypeStruct((N, D), jnp.float32),
    mesh=_mesh,
    scratch_types=[
        pltpu.VMEM((RCH, D), jnp.float32),
        pltpu.SemaphoreType.DMA,
    ],
)


def kernel(x_e, graph_ids, entity_ids):
    graph_ids = graph_ids.astype(jnp.int32)
    entity_ids = entity_ids.astype(jnp.int32)
    parts, rows = _main(x_e, entity_ids, graph_ids)
    bad = jnp.sum(parts) != 0
    return lax.cond(bad, lambda r: _nanfill(), lambda r: r, rows)
